# baseline (speedup 1.0000x reference)
import functools

import jax
import jax.numpy as jnp
from jax import lax
from jax.experimental import pallas as pl
from jax.experimental.pallas import tpu as pltpu
from jax.experimental.pallas import tpu_sc as plsc

_N = 10000
_E = 320000
_D = 128
_SCALE = 1.8

_NC = 2
_NS = 16
_NW = _NC * _NS
_EPT = _E // _NW
_CH = 80
_NCH = _EPT // _CH
_CHP = 64
_EPTP = 10240
_NCHP = _EPTP // _CHP
_JUNK = 10239

_NPAD = 10240
_DEGW = 16

_ROWS_PT = _NPAD // _NS
_ZCH = 128


def _deg_body(dst2, zeros_hbm, deg_out, dst_v, hist_v, blk_v, hist_sh):
    c = lax.axis_index("c")
    s = lax.axis_index("s")
    wid = c * _NS + s
    pltpu.sync_copy(zeros_hbm, hist_v)
    pltpu.sync_copy(dst2.at[wid], dst_v)
    ones = jnp.full((16,), 1.0, jnp.float32)

    def step(i, carry):
        ix = dst_v[pl.ds(i * 16, 16)]
        plsc.addupdate_scatter(hist_v, [ix], ones)
        return carry

    lax.fori_loop(0, _EPT // 16, step, 0)
    pltpu.sync_copy(hist_v, hist_sh.at[s])
    plsc.subcore_barrier()
    sl = pl.ds(s * _ROWS_PT, _ROWS_PT)
    for k in range(_NS):
        pltpu.sync_copy(hist_sh.at[k, sl], blk_v.at[k])

    def red(i, carry):
        acc = blk_v[0, pl.ds(i * 16, 16)]
        for k in range(1, _NS):
            acc = acc + blk_v[k, pl.ds(i * 16, 16)]
        hist_v[pl.ds(i * 16, 16)] = acc
        return carry

    lax.fori_loop(0, _ROWS_PT // 16, red, 0)
    pltpu.sync_copy(hist_v.at[pl.ds(0, _ROWS_PT)], deg_out.at[c, sl])


def _deg_hist(dst2, zeros_npad):
    mesh = plsc.VectorSubcoreMesh(core_axis_name="c", subcore_axis_name="s", num_cores=_NC, num_subcores=_NS)
    return pl.kernel(
        _deg_body,
        out_type=jax.ShapeDtypeStruct((_NC, _NPAD), jnp.float32),
        mesh=mesh,
        compiler_params=pltpu.CompilerParams(needs_layout_passes=False),
        scratch_types=[
            pltpu.VMEM((_EPT,), jnp.int32),
            pltpu.VMEM((_NPAD,), jnp.float32),
            pltpu.VMEM((_NS, _ROWS_PT), jnp.float32),
            pltpu.VMEM_SHARED((_NS, _NPAD), jnp.float32),
        ],
    )(dst2, zeros_npad)


def _linear_body(x_ref, w_ref, b_ref, deg_ref, g_ref):
    h = lax.dot_general(
        x_ref[...], w_ref[...], (((1,), (1,)), ((), ())),
        preferred_element_type=jnp.float32,
    ) + b_ref[...]
    nrm = jnp.sqrt(jnp.sum(h * h, axis=1, keepdims=True))
    hn = h * (_SCALE / jnp.maximum(nrm, 1e-12))
    d = deg_ref[0, :_N] + deg_ref[1, :_N]
    dinv = lax.rsqrt(jnp.maximum(d + 1.0, 1.0))
    g_ref[...] = hn * dinv


def _linear(x, W1, b1, deg):
    return pl.pallas_call(
        _linear_body,
        out_shape=jax.ShapeDtypeStruct((_N, _D), jnp.float32),
    )(x, W1, b1.reshape(1, _D), deg)


def _prop_body(g_hbm, ed_hbm, zeros_hbm, s_out,
               i0, i1, i2, i3, rows_a, rows_b, s_sh,
               gsa, gsb, is0, is1, is2, is3):
    slots = (i0, i1, i2, i3)
    isems = (is0, is1, is2, is3)
    c = lax.axis_index("c")
    s = lax.axis_index("s")
    wid = c * _NS + s
    for k in range(_ROWS_PT // _ZCH):
        pltpu.sync_copy(
            zeros_hbm, s_sh.at[pl.ds(s * _ROWS_PT + k * _ZCH, _ZCH)])

    def idx_copy(j, k):
        pltpu.async_copy(ed_hbm.at[wid, j], slots[k], isems[k])

    def wait_idx(k):
        pltpu.make_async_copy(ed_hbm.at[wid, 0], slots[k], isems[k]).wait()

    def gather(rows, k, gsem):
        pltpu.async_copy(g_hbm.at[slots[k].at[0]], rows, gsem)

    def wait_gather(rows, k, gsem):
        pltpu.make_async_copy(g_hbm.at[slots[k].at[0]], rows, gsem).wait()

    def scatter(rows, k):
        pltpu.sync_copy(rows, s_sh.at[slots[k].at[1]], add=True)

    for k in range(4):
        idx_copy(k, k)
    plsc.subcore_barrier()
    wait_idx(0)
    gather(rows_a, 0, gsa)
    wait_idx(1)
    gather(rows_b, 1, gsb)

    def sub(j, k, rows, gsem):
        wait_gather(rows, k, gsem)
        scatter(rows, k)

        @pl.when(j + 4 < _NCHP)
        def _():
            idx_copy(j + 4, k)

        @pl.when(j + 2 < _NCHP)
        def _():
            wait_idx((k + 2) % 4)
            gather(rows, (k + 2) % 4, gsem)

    def quad(i, carry):
        j = 4 * i
        sub(j, 0, rows_a, gsa)
        sub(j + 1, 1, rows_b, gsb)
        sub(j + 2, 2, rows_a, gsa)
        sub(j + 3, 3, rows_b, gsb)
        return carry

    lax.fori_loop(0, _NCHP // 4, quad, 0)
    plsc.subcore_barrier()
    sl = pl.ds(s * _ROWS_PT, _ROWS_PT)
    pltpu.sync_copy(s_sh.at[sl], s_out.at[c, sl])


def _propagate(g, ed, zrows):
    mesh = plsc.VectorSubcoreMesh(core_axis_name="c", subcore_axis_name="s", num_cores=_NC, num_subcores=_NS)
    return pl.kernel(
        _prop_body,
        out_type=jax.ShapeDtypeStruct((_NC, _NPAD, _D), jnp.float32),
        mesh=mesh,
        scratch_types=[
            pltpu.VMEM((2, _CHP), jnp.int32),
            pltpu.VMEM((2, _CHP), jnp.int32),
            pltpu.VMEM((2, _CHP), jnp.int32),
            pltpu.VMEM((2, _CHP), jnp.int32),
            pltpu.VMEM((_CHP, _D), jnp.float32),
            pltpu.VMEM((_CHP, _D), jnp.float32),
            pltpu.VMEM_SHARED((_NPAD, _D), jnp.float32),
            pltpu.SemaphoreType.DMA,
            pltpu.SemaphoreType.DMA,
            pltpu.SemaphoreType.DMA,
            pltpu.SemaphoreType.DMA,
            pltpu.SemaphoreType.DMA,
            pltpu.SemaphoreType.DMA,
        ],
    )(g, ed, zrows)


def _combine_body(g_ref, s_ref, deg_ref, o_ref):
    d = deg_ref[0, :_N] + deg_ref[1, :_N]
    dinv = lax.rsqrt(jnp.maximum(d + 1.0, 1.0))
    o_ref[...] = dinv * (g_ref[...] + s_ref[0, :_N] + s_ref[1, :_N])


def _combine(g, s_part, deg):
    return pl.pallas_call(
        _combine_body,
        out_shape=jax.ShapeDtypeStruct((_N, _D), jnp.float32),
    )(g, s_part, deg)


def kernel(x, edge_index, W1, b1):
    pad = _EPTP - _EPT
    src2 = jnp.pad(edge_index[0].reshape(_NW, _EPT), ((0, 0), (0, pad)))
    dst2p = jnp.pad(edge_index[1].reshape(_NW, _EPT), ((0, 0), (0, pad)),
                    constant_values=_JUNK)
    ed = jnp.stack([src2.reshape(_NW, _NCHP, _CHP),
                    dst2p.reshape(_NW, _NCHP, _CHP)], axis=2)
    dst2 = edge_index[1].reshape(_NW, _EPT)
    zeros_npad = jnp.zeros((_NPAD,), jnp.float32)
    zrows = jnp.zeros((_ZCH, _D), jnp.float32)

    deg = _deg_hist(dst2, zeros_npad).reshape(_NC, _NPAD, 1)
    g = _linear(x, W1, b1, deg)
    s_part = _propagate(g, ed, zrows)
    return _combine(g, s_part, deg)


# speedup vs baseline: 2.4701x; 2.4701x over previous
import functools

import jax
import jax.numpy as jnp
from jax import lax
from jax.experimental import pallas as pl
from jax.experimental.pallas import tpu as pltpu
from jax.experimental.pallas import tpu_sc as plsc

_N = 10000
_E = 320000
_D = 128
_SCALE = 1.8

_NC = 2
_NS = 16
_NW = _NC * _NS
_EPT = _E // _NW
_CH = 80
_NCH = _EPT // _CH

_NPAD = 10240
_DEGW = 16

_ROWS_PT = _NPAD // _NS
_ZCH = 128


def _deg_body(dst2, zeros_hbm, deg_out, dst_v, hist_v, blk_v, hist_sh):
    c = lax.axis_index("c")
    s = lax.axis_index("s")
    wid = c * _NS + s
    pltpu.sync_copy(zeros_hbm, hist_v)
    pltpu.sync_copy(dst2.at[wid], dst_v)
    ones = jnp.full((16,), 1.0, jnp.float32)

    def step(i, carry):
        ix = dst_v[pl.ds(i * 16, 16)]
        plsc.addupdate_scatter(hist_v, [ix], ones)
        return carry

    lax.fori_loop(0, _EPT // 16, step, 0)
    pltpu.sync_copy(hist_v, hist_sh.at[s])
    plsc.subcore_barrier()
    sl = pl.ds(s * _ROWS_PT, _ROWS_PT)
    for k in range(_NS):
        pltpu.sync_copy(hist_sh.at[k, sl], blk_v.at[k])

    def red(i, carry):
        acc = blk_v[0, pl.ds(i * 16, 16)]
        for k in range(1, _NS):
            acc = acc + blk_v[k, pl.ds(i * 16, 16)]
        hist_v[pl.ds(i * 16, 16)] = acc
        return carry

    lax.fori_loop(0, _ROWS_PT // 16, red, 0)
    pltpu.sync_copy(hist_v.at[pl.ds(0, _ROWS_PT)], deg_out.at[c, sl])


def _deg_hist(dst2, zeros_npad):
    mesh = plsc.VectorSubcoreMesh(core_axis_name="c", subcore_axis_name="s", num_cores=_NC, num_subcores=_NS)
    return pl.kernel(
        _deg_body,
        out_type=jax.ShapeDtypeStruct((_NC, _NPAD), jnp.float32),
        mesh=mesh,
        compiler_params=pltpu.CompilerParams(needs_layout_passes=False),
        scratch_types=[
            pltpu.VMEM((_EPT,), jnp.int32),
            pltpu.VMEM((_NPAD,), jnp.float32),
            pltpu.VMEM((_NS, _ROWS_PT), jnp.float32),
            pltpu.VMEM_SHARED((_NS, _NPAD), jnp.float32),
        ],
    )(dst2, zeros_npad)


def _linear_body(x_ref, w_ref, b_ref, deg_ref, g_ref):
    h = lax.dot_general(
        x_ref[...], w_ref[...], (((1,), (1,)), ((), ())),
        preferred_element_type=jnp.float32,
    ) + b_ref[...]
    nrm = jnp.sqrt(jnp.sum(h * h, axis=1, keepdims=True))
    hn = h * (_SCALE / jnp.maximum(nrm, 1e-12))
    d = deg_ref[0, :_N] + deg_ref[1, :_N]
    dinv = lax.rsqrt(jnp.maximum(d + 1.0, 1.0))
    g_ref[...] = hn * dinv


def _linear(x, W1, b1, deg):
    return pl.pallas_call(
        _linear_body,
        out_shape=jax.ShapeDtypeStruct((_N, _D), jnp.float32),
    )(x, W1, b1.reshape(1, _D), deg)


def _prop_body(g_hbm, ed_hbm, zeros_hbm, s_out, ed_v,
               sa_a, da_a, sa_b, da_b, rows_a, rows_b, s_sh, gsa, gsb):
    c = lax.axis_index("c")
    s = lax.axis_index("s")
    wid = c * _NS + s
    for k in range(_ROWS_PT // _ZCH):
        pltpu.sync_copy(
            zeros_hbm, s_sh.at[pl.ds(s * _ROWS_PT + k * _ZCH, _ZCH)])
    pltpu.sync_copy(ed_hbm.at[wid], ed_v)
    plsc.subcore_barrier()

    def unpack(j, sa, da):
        for k in range(_CH // 16):
            v = ed_v[j, pl.ds(k * 16, 16)]
            sa[pl.ds(k * 16, 16)] = lax.shift_right_logical(v, 16)
            da[pl.ds(k * 16, 16)] = lax.bitwise_and(v, 0xFFFF)

    def gather(rows, sa, sem):
        pltpu.async_copy(g_hbm.at[sa], rows, sem)

    def wait_gather(rows, sa, sem):
        pltpu.make_async_copy(g_hbm.at[sa], rows, sem).wait()

    def scatter(rows, da):
        pltpu.sync_copy(rows, s_sh.at[da], add=True)

    unpack(0, sa_a, da_a)
    gather(rows_a, sa_a, gsa)

    def pair(i, carry):
        j = 2 * i
        unpack(j + 1, sa_b, da_b)
        gather(rows_b, sa_b, gsb)
        wait_gather(rows_a, sa_a, gsa)
        scatter(rows_a, da_a)
        unpack(j + 2, sa_a, da_a)
        gather(rows_a, sa_a, gsa)
        wait_gather(rows_b, sa_b, gsb)
        scatter(rows_b, da_b)
        return carry

    lax.fori_loop(0, (_NCH - 1) // 2, pair, 0)
    wait_gather(rows_a, sa_a, gsa)
    scatter(rows_a, da_a)
    plsc.subcore_barrier()
    sl = pl.ds(s * _ROWS_PT, _ROWS_PT)
    pltpu.sync_copy(s_sh.at[sl], s_out.at[c, sl])


def _propagate(g, ed, zrows):
    mesh = plsc.VectorSubcoreMesh(core_axis_name="c", subcore_axis_name="s", num_cores=_NC, num_subcores=_NS)
    return pl.kernel(
        _prop_body,
        out_type=jax.ShapeDtypeStruct((_NC, _NPAD, _D), jnp.float32),
        mesh=mesh,
        compiler_params=pltpu.CompilerParams(needs_layout_passes=False),
        scratch_types=[
            pltpu.VMEM((_NCH, _CH), jnp.int32),
            pltpu.VMEM((_CH,), jnp.int32),
            pltpu.VMEM((_CH,), jnp.int32),
            pltpu.VMEM((_CH,), jnp.int32),
            pltpu.VMEM((_CH,), jnp.int32),
            pltpu.VMEM((_CH, _D), jnp.float32),
            pltpu.VMEM((_CH, _D), jnp.float32),
            pltpu.VMEM_SHARED((_NPAD, _D), jnp.float32),
            pltpu.SemaphoreType.DMA,
            pltpu.SemaphoreType.DMA,
        ],
    )(g, ed, zrows)


def _combine_body(g_ref, s_ref, deg_ref, o_ref):
    d = deg_ref[0, :_N] + deg_ref[1, :_N]
    dinv = lax.rsqrt(jnp.maximum(d + 1.0, 1.0))
    o_ref[...] = dinv * (g_ref[...] + s_ref[0, :_N] + s_ref[1, :_N])


def _combine(g, s_part, deg):
    return pl.pallas_call(
        _combine_body,
        out_shape=jax.ShapeDtypeStruct((_N, _D), jnp.float32),
    )(g, s_part, deg)


def kernel(x, edge_index, W1, b1):
    packed = jnp.bitwise_or(jnp.left_shift(edge_index[0], 16),
                            edge_index[1])
    ed = packed.reshape(_NW, _NCH, _CH)
    dst2 = edge_index[1].reshape(_NW, _EPT)
    zeros_npad = jnp.zeros((_NPAD,), jnp.float32)
    zrows = jnp.zeros((_ZCH, _D), jnp.float32)

    deg = _deg_hist(dst2, zeros_npad).reshape(_NC, _NPAD, 1)
    g = _linear(x, W1, b1, deg)
    s_part = _propagate(g, ed, zrows)
    return _combine(g, s_part, deg)


# speedup vs baseline: 2.4702x; 1.0000x over previous
import functools

import jax
import jax.numpy as jnp
from jax import lax
from jax.experimental import pallas as pl
from jax.experimental.pallas import tpu as pltpu
from jax.experimental.pallas import tpu_sc as plsc

_N = 10000
_E = 320000
_D = 128
_SCALE = 1.8

_NC = 2
_NS = 16
_NW = _NC * _NS
_EPT = _E // _NW
_CH = 80
_NCH = _EPT // _CH

_NPAD = 10240
_DEGW = 16

_ROWS_PT = _NPAD // _NS
_ZCH = 128


def _deg_body(dst2, zeros_hbm, deg_out, dst_v, hist_v, blk_v, hist_sh):
    c = lax.axis_index("c")
    s = lax.axis_index("s")
    wid = c * _NS + s
    pltpu.sync_copy(zeros_hbm, hist_v)
    pltpu.sync_copy(dst2.at[wid], dst_v)
    ones = jnp.full((16,), 1.0, jnp.float32)

    def step(i, carry):
        for u in range(5):
            ix = dst_v[pl.ds((5 * i + u) * 16, 16)]
            plsc.addupdate_scatter(hist_v, [ix], ones)
        return carry

    lax.fori_loop(0, _EPT // 80, step, 0)
    pltpu.sync_copy(hist_v, hist_sh.at[s])
    plsc.subcore_barrier()
    sl = pl.ds(s * _ROWS_PT, _ROWS_PT)
    for k in range(_NS):
        pltpu.sync_copy(hist_sh.at[k, sl], blk_v.at[k])

    def red(i, carry):
        acc = blk_v[0, pl.ds(i * 16, 16)]
        for k in range(1, _NS):
            acc = acc + blk_v[k, pl.ds(i * 16, 16)]
        hist_v[pl.ds(i * 16, 16)] = acc
        return carry

    lax.fori_loop(0, _ROWS_PT // 16, red, 0)
    pltpu.sync_copy(hist_v.at[pl.ds(0, _ROWS_PT)], deg_out.at[c, sl])


def _deg_hist(dst2, zeros_npad):
    mesh = plsc.VectorSubcoreMesh(core_axis_name="c", subcore_axis_name="s", num_cores=_NC, num_subcores=_NS)
    return pl.kernel(
        _deg_body,
        out_type=jax.ShapeDtypeStruct((_NC, _NPAD), jnp.float32),
        mesh=mesh,
        compiler_params=pltpu.CompilerParams(needs_layout_passes=False),
        scratch_types=[
            pltpu.VMEM((_EPT,), jnp.int32),
            pltpu.VMEM((_NPAD,), jnp.float32),
            pltpu.VMEM((_NS, _ROWS_PT), jnp.float32),
            pltpu.VMEM_SHARED((_NS, _NPAD), jnp.float32),
        ],
    )(dst2, zeros_npad)


def _linear_body(x_ref, w_ref, b_ref, deg_ref, g_ref):
    h = lax.dot_general(
        x_ref[...], w_ref[...], (((1,), (1,)), ((), ())),
        preferred_element_type=jnp.float32,
    ) + b_ref[...]
    nrm = jnp.sqrt(jnp.sum(h * h, axis=1, keepdims=True))
    hn = h * (_SCALE / jnp.maximum(nrm, 1e-12))
    d = deg_ref[0, :_N] + deg_ref[1, :_N]
    dinv = lax.rsqrt(jnp.maximum(d + 1.0, 1.0))
    g_ref[...] = hn * dinv


def _linear(x, W1, b1, deg):
    return pl.pallas_call(
        _linear_body,
        out_shape=jax.ShapeDtypeStruct((_N, _D), jnp.float32),
    )(x, W1, b1.reshape(1, _D), deg)


def _prop_body(g_hbm, ed_hbm, zeros_hbm, s_out, ed_v,
               sa_a, da_a, sa_b, da_b, rows_a, rows_b, s_sh, gsa, gsb):
    c = lax.axis_index("c")
    s = lax.axis_index("s")
    wid = c * _NS + s
    for k in range(_ROWS_PT // _ZCH):
        pltpu.sync_copy(
            zeros_hbm, s_sh.at[pl.ds(s * _ROWS_PT + k * _ZCH, _ZCH)])
    pltpu.sync_copy(ed_hbm.at[wid], ed_v)
    plsc.subcore_barrier()

    def unpack(j, sa, da):
        for k in range(_CH // 16):
            v = ed_v[j, pl.ds(k * 16, 16)]
            sa[pl.ds(k * 16, 16)] = lax.shift_right_logical(v, 16)
            da[pl.ds(k * 16, 16)] = lax.bitwise_and(v, 0xFFFF)

    def gather(rows, sa, sem):
        pltpu.async_copy(g_hbm.at[sa], rows, sem)

    def wait_gather(rows, sa, sem):
        pltpu.make_async_copy(g_hbm.at[sa], rows, sem).wait()

    def scatter(rows, da):
        pltpu.sync_copy(rows, s_sh.at[da], add=True)

    unpack(0, sa_a, da_a)
    gather(rows_a, sa_a, gsa)

    def pair(i, carry):
        j = 2 * i
        unpack(j + 1, sa_b, da_b)
        gather(rows_b, sa_b, gsb)
        wait_gather(rows_a, sa_a, gsa)
        scatter(rows_a, da_a)
        unpack(j + 2, sa_a, da_a)
        gather(rows_a, sa_a, gsa)
        wait_gather(rows_b, sa_b, gsb)
        scatter(rows_b, da_b)
        return carry

    lax.fori_loop(0, (_NCH - 1) // 2, pair, 0)
    wait_gather(rows_a, sa_a, gsa)
    scatter(rows_a, da_a)
    plsc.subcore_barrier()
    sl = pl.ds(s * _ROWS_PT, _ROWS_PT)
    pltpu.sync_copy(s_sh.at[sl], s_out.at[c, sl])


def _propagate(g, ed, zrows):
    mesh = plsc.VectorSubcoreMesh(core_axis_name="c", subcore_axis_name="s", num_cores=_NC, num_subcores=_NS)
    return pl.kernel(
        _prop_body,
        out_type=jax.ShapeDtypeStruct((_NC, _NPAD, _D), jnp.float32),
        mesh=mesh,
        compiler_params=pltpu.CompilerParams(needs_layout_passes=False),
        scratch_types=[
            pltpu.VMEM((_NCH, _CH), jnp.int32),
            pltpu.VMEM((_CH,), jnp.int32),
            pltpu.VMEM((_CH,), jnp.int32),
            pltpu.VMEM((_CH,), jnp.int32),
            pltpu.VMEM((_CH,), jnp.int32),
            pltpu.VMEM((_CH, _D), jnp.float32),
            pltpu.VMEM((_CH, _D), jnp.float32),
            pltpu.VMEM_SHARED((_NPAD, _D), jnp.float32),
            pltpu.SemaphoreType.DMA,
            pltpu.SemaphoreType.DMA,
        ],
    )(g, ed, zrows)


def _combine_body(g_ref, s_ref, deg_ref, o_ref):
    d = deg_ref[0, :_N] + deg_ref[1, :_N]
    dinv = lax.rsqrt(jnp.maximum(d + 1.0, 1.0))
    o_ref[...] = dinv * (g_ref[...] + s_ref[0, :_N] + s_ref[1, :_N])


def _combine(g, s_part, deg):
    return pl.pallas_call(
        _combine_body,
        out_shape=jax.ShapeDtypeStruct((_N, _D), jnp.float32),
    )(g, s_part, deg)


def kernel(x, edge_index, W1, b1):
    packed = jnp.bitwise_or(jnp.left_shift(edge_index[0], 16),
                            edge_index[1])
    ed = packed.reshape(_NW, _NCH, _CH)
    dst2 = edge_index[1].reshape(_NW, _EPT)
    zeros_npad = jnp.zeros((_NPAD,), jnp.float32)
    zrows = jnp.zeros((_ZCH, _D), jnp.float32)

    deg = _deg_hist(dst2, zeros_npad).reshape(_NC, _NPAD, 1)
    g = _linear(x, W1, b1, deg)
    s_part = _propagate(g, ed, zrows)
    return _combine(g, s_part, deg)
